# Initial kernel scaffold; baseline (speedup 1.0000x reference)
import jax
import jax.numpy as jnp
from jax.experimental import pallas as pl


def kernel(x, edge_index, W1, b1, W2, b2):
    raise NotImplementedError("write your pallas kernel here")


# speedup vs baseline: 7.6137x; 7.6137x over previous
import dataclasses
import functools

import jax
import jax.numpy as jnp
from jax import lax
from jax.experimental import pallas as pl
from jax.experimental.pallas import tpu as pltpu
from jax.experimental.pallas import tpu_sc as plsc

N = 100000
E = 1600000
D = 48
K_ITERS = 16
ALPHA_C = 0.1

BLK = 12800
NBLK = 8
NPAD = BLK * NBLK
SLOTS = 4
TPB = BLK // 16
G = 512
RB = 2000
NROW_BLOCKS = N // RB

_mesh = plsc.VectorSubcoreMesh(core_axis_name="c", subcore_axis_name="s")

_sc_params = pltpu.CompilerParams()
for _f, _v in (("needs_layout_passes", False), ("use_tc_tiling_on_sc", False)):
  if _f in pltpu.CompilerParams.__dataclass_fields__:
    _sc_params = dataclasses.replace(_sc_params, **{_f: _v})


def _clean_indices(srcv, dstv, csrc, cdst, lane, base, my_lo, my_hi, lo_row):
  for gb in range(G // 16):
    sl = pl.ds(gb * 16, 16)
    d16 = dstv[sl]
    s16 = srcv[sl]
    eid = lane + (base + gb * 16)
    m = (eid >= my_lo) & (eid < my_hi)
    csrc[sl] = jnp.where(m, s16, 0)
    cdst[sl] = jnp.where(m, d16 - lo_row, jnp.int32(BLK))


def _block_bounds(bvec_ref, lane, blk_id):
  neg = jnp.int32(-(2**31))
  b16 = bvec_ref[...]
  e_lo = jnp.max(jnp.where(lane == blk_id, b16, neg))
  e_hi = jnp.max(jnp.where(lane == blk_id + 1, b16, neg))
  return e_lo, e_hi


def _my_range(e_lo, e_hi, s):
  m_len = e_hi - e_lo
  my_lo = e_lo + (m_len * s) // 16
  my_hi = e_lo + (m_len * (s + 1)) // 16
  a_lo = (my_lo // 8) * 8
  nch = (my_hi - a_lo + (G - 1)) // G
  return my_lo, my_hi, a_lo, nch


def _prop_sc(u, src_s, dst_s, bounds):
  @functools.partial(
      pl.kernel,
      out_type=jax.ShapeDtypeStruct((NPAD, D), jnp.float32),
      mesh=_mesh,
      compiler_params=_sc_params,
      scratch_types=[
          pltpu.VMEM_SHARED((BLK + 8, D), jnp.float32),
          pltpu.VMEM((TPB // 5, D), jnp.float32),
          pltpu.VMEM((G,), jnp.int32),
          pltpu.VMEM((G,), jnp.int32),
          pltpu.VMEM((G,), jnp.int32),
          pltpu.VMEM((G,), jnp.int32),
          pltpu.VMEM((G, D), jnp.float32),
          pltpu.VMEM((16,), jnp.int32),
      ],
  )
  def k(u_hbm, src_hbm, dst_hbm, bnd_hbm, agg_hbm,
        acc, zbuf, srcv, dstv, csrc, cdst, rows, bvec):
    c = lax.axis_index("c")
    s = lax.axis_index("s")
    lane = lax.broadcasted_iota(jnp.int32, (16,), 0)
    zrows = TPB // 5

    @pl.loop(0, zrows)
    def _(r):
      for j in range(D // 16):
        zbuf[r, pl.ds(j * 16, 16)] = jnp.zeros((16,), jnp.float32)

    pltpu.sync_copy(bnd_hbm, bvec)

    for slot in range(SLOTS):
      blk_id = c * SLOTS + slot
      lo_row = blk_id * BLK
      e_lo, e_hi = _block_bounds(bvec, lane, blk_id)
      my_lo, my_hi, a_lo, nch = _my_range(e_lo, e_hi, s)

      for z in range(TPB // zrows):
        pltpu.sync_copy(zbuf, acc.at[pl.ds(s * TPB + z * zrows, zrows)])
      plsc.subcore_barrier()

      def chunk(i, carry):
        base = a_lo + i * G
        pltpu.sync_copy(src_hbm.at[pl.ds(base, G)], srcv)
        pltpu.sync_copy(dst_hbm.at[pl.ds(base, G)], dstv)
        _clean_indices(srcv, dstv, csrc, cdst, lane, base, my_lo, my_hi,
                       lo_row)
        pltpu.sync_copy(u_hbm.at[csrc], rows)
        pltpu.sync_copy(rows, acc.at[cdst], add=True)
        return carry

      lax.fori_loop(0, nch, chunk, 0)
      plsc.subcore_barrier()
      pltpu.sync_copy(acc.at[pl.ds(s * TPB, TPB)],
                      agg_hbm.at[pl.ds(lo_row + s * TPB, TPB)])

  return k(u, src_s, dst_s, bounds)


def _deg_sc(dst_s, bounds):
  @functools.partial(
      pl.kernel,
      out_type=jax.ShapeDtypeStruct((NPAD, 16), jnp.float32),
      mesh=_mesh,
      compiler_params=_sc_params,
      scratch_types=[
          pltpu.VMEM_SHARED((BLK + 8, 16), jnp.float32),
          pltpu.VMEM((G, 16), jnp.float32),
          pltpu.VMEM((TPB // 5, 16), jnp.float32),
          pltpu.VMEM((G,), jnp.int32),
          pltpu.VMEM((G,), jnp.int32),
          pltpu.VMEM((G,), jnp.int32),
          pltpu.VMEM((16,), jnp.int32),
      ],
  )
  def k(dst_hbm, bnd_hbm, deg_hbm, acc, ones, zbuf, srcv, dstv, cdst, bvec):
    c = lax.axis_index("c")
    s = lax.axis_index("s")
    lane = lax.broadcasted_iota(jnp.int32, (16,), 0)
    zrows = TPB // 5

    @pl.loop(0, G)
    def _(r):
      ones[r, pl.ds(0, 16)] = jnp.ones((16,), jnp.float32)

    @pl.loop(0, zrows)
    def _(r):
      zbuf[r, pl.ds(0, 16)] = jnp.zeros((16,), jnp.float32)

    pltpu.sync_copy(bnd_hbm, bvec)

    for slot in range(SLOTS):
      blk_id = c * SLOTS + slot
      lo_row = blk_id * BLK
      e_lo, e_hi = _block_bounds(bvec, lane, blk_id)
      my_lo, my_hi, a_lo, nch = _my_range(e_lo, e_hi, s)

      for z in range(TPB // zrows):
        pltpu.sync_copy(zbuf, acc.at[pl.ds(s * TPB + z * zrows, zrows)])
      plsc.subcore_barrier()

      def chunk(i, carry):
        base = a_lo + i * G
        pltpu.sync_copy(dst_hbm.at[pl.ds(base, G)], dstv)
        _clean_indices(dstv, dstv, srcv, cdst, lane, base, my_lo, my_hi,
                       lo_row)
        pltpu.sync_copy(ones, acc.at[cdst], add=True)
        return carry

      lax.fori_loop(0, nch, chunk, 0)
      plsc.subcore_barrier()
      pltpu.sync_copy(acc.at[pl.ds(s * TPB, TPB)],
                      deg_hbm.at[pl.ds(lo_row + s * TPB, TPB)])

  return k(dst_s, bounds)


def _mlp_tc(x, w1t, b1, w2t, b2):
  def body(x_ref, w1_ref, b1_ref, w2_ref, b2_ref, o_ref):
    h1 = jnp.dot(x_ref[...], w1_ref[...],
                 preferred_element_type=jnp.float32) + b1_ref[0:1, :]
    h1 = jnp.maximum(h1, 0.0)
    o_ref[...] = jnp.dot(h1, w2_ref[...],
                         preferred_element_type=jnp.float32) + b2_ref[0:1, :]

  return pl.pallas_call(
      body,
      grid=(NROW_BLOCKS,),
      in_specs=[
          pl.BlockSpec((RB, 128), lambda i: (i, 0)),
          pl.BlockSpec((128, 128), lambda i: (0, 0)),
          pl.BlockSpec((8, 128), lambda i: (0, 0)),
          pl.BlockSpec((128, D), lambda i: (0, 0)),
          pl.BlockSpec((8, D), lambda i: (0, 0)),
      ],
      out_specs=pl.BlockSpec((RB, D), lambda i: (i, 0)),
      out_shape=jax.ShapeDtypeStruct((N, D), jnp.float32),
  )(x, w1t, b1, w2t, b2)


def _prep_tc(h, deg16):
  def body(h_ref, d_ref, u_ref, c_ref, g_ref, q_ref):
    deg = d_ref[:, 0:1] + 1.0
    dinv = lax.rsqrt(deg)
    u0 = h_ref[...] * dinv
    u_ref[...] = u0
    c_ref[...] = jnp.broadcast_to((1.0 - ALPHA_C) * dinv * dinv,
                                  (RB, D))
    g_ref[...] = ALPHA_C * u0
    q_ref[...] = jnp.broadcast_to(deg * dinv, (RB, D))

  return pl.pallas_call(
      body,
      grid=(NROW_BLOCKS,),
      in_specs=[
          pl.BlockSpec((RB, D), lambda i: (i, 0)),
          pl.BlockSpec((RB, 16), lambda i: (i, 0)),
      ],
      out_specs=[
          pl.BlockSpec((RB, D), lambda i: (i, 0)),
          pl.BlockSpec((RB, D), lambda i: (i, 0)),
          pl.BlockSpec((RB, D), lambda i: (i, 0)),
          pl.BlockSpec((RB, D), lambda i: (i, 0)),
      ],
      out_shape=[jax.ShapeDtypeStruct((N, D), jnp.float32)] * 4,
  )(h, deg16)


def _update_tc(agg, u, c, g):
  def body(a_ref, u_ref, c_ref, g_ref, o_ref):
    o_ref[...] = c_ref[...] * (a_ref[...] + u_ref[...]) + g_ref[...]

  return pl.pallas_call(
      body,
      grid=(NROW_BLOCKS,),
      in_specs=[pl.BlockSpec((RB, D), lambda i: (i, 0))] * 4,
      out_specs=pl.BlockSpec((RB, D), lambda i: (i, 0)),
      out_shape=jax.ShapeDtypeStruct((N, D), jnp.float32),
  )(agg, u, c, g)


def _final_tc(u, dsq):
  def body(u_ref, q_ref, o_ref):
    z = u_ref[...] * q_ref[...]
    m = jnp.max(z, axis=1, keepdims=True)
    e = jnp.exp(z - m)
    ssum = jnp.sum(e, axis=1, keepdims=True)
    o_ref[...] = z - m - jnp.log(ssum)

  return pl.pallas_call(
      body,
      grid=(NROW_BLOCKS,),
      in_specs=[pl.BlockSpec((RB, D), lambda i: (i, 0))] * 2,
      out_specs=pl.BlockSpec((RB, D), lambda i: (i, 0)),
      out_shape=jax.ShapeDtypeStruct((N, D), jnp.float32),
  )(u, dsq)


def kernel(x, edge_index, W1, b1, W2, b2):
  src = edge_index[0]
  dst = edge_index[1]
  order = jnp.argsort(dst)
  src_s = jnp.concatenate([src[order], jnp.zeros((G,), jnp.int32)])
  dst_s = jnp.concatenate([dst[order], jnp.zeros((G,), jnp.int32)])
  edges = jnp.arange(16, dtype=jnp.int32) * BLK
  edges = jnp.minimum(edges, N)
  bounds = jnp.searchsorted(dst_s[:E], edges).astype(jnp.int32)

  h = _mlp_tc(x, W1.T, jnp.tile(b1[None, :], (8, 1)),
              W2.T, jnp.tile(b2[None, :], (8, 1)))
  deg16 = _deg_sc(dst_s, bounds)
  u, c, g, dsq = _prep_tc(h, deg16[:N])

  def step(_, u):
    agg = _prop_sc(u, src_s, dst_s, bounds)
    return _update_tc(agg[:N], u, c, g)

  u = lax.fori_loop(0, K_ITERS, step, u)
  return _final_tc(u, dsq)


# speedup vs baseline: 8.0218x; 1.0536x over previous
import dataclasses
import functools

import jax
import jax.numpy as jnp
from jax import lax
from jax.experimental import pallas as pl
from jax.experimental.pallas import tpu as pltpu
from jax.experimental.pallas import tpu_sc as plsc

N = 100000
E = 1600000
D = 48
K_ITERS = 16
ALPHA_C = 0.1

BLK = 12800
NBLK = 8
NPAD = BLK * NBLK
SLOTS = 4
TPB = BLK // 16
G = 512
RB = 2000
NROW_BLOCKS = N // RB

_mesh = plsc.VectorSubcoreMesh(core_axis_name="c", subcore_axis_name="s")

_sc_params = pltpu.CompilerParams()
for _f, _v in (("needs_layout_passes", False), ("use_tc_tiling_on_sc", False)):
  if _f in pltpu.CompilerParams.__dataclass_fields__:
    _sc_params = dataclasses.replace(_sc_params, **{_f: _v})


def _clean_indices(srcv, dstv, csrc, cdst, lane, base, my_lo, my_hi, lo_row,
                   b=None):
  for gb in range(G // 16):
    sl = pl.ds(gb * 16, 16)
    if b is None:
      d16 = dstv[sl]
      s16 = srcv[sl]
    else:
      d16 = dstv[b, sl]
      s16 = srcv[b, sl]
    eid = lane + (base + gb * 16)
    m = (eid >= my_lo) & (eid < my_hi)
    if b is None:
      csrc[sl] = jnp.where(m, s16, 0)
      cdst[sl] = jnp.where(m, d16 - lo_row, jnp.int32(BLK))
    else:
      csrc[b, sl] = jnp.where(m, s16, 0)
      cdst[b, sl] = jnp.where(m, d16 - lo_row, jnp.int32(BLK))


def _block_bounds(bvec_ref, lane, blk_id):
  neg = jnp.int32(-(2**31))
  b16 = bvec_ref[...]
  e_lo = jnp.max(jnp.where(lane == blk_id, b16, neg))
  e_hi = jnp.max(jnp.where(lane == blk_id + 1, b16, neg))
  return e_lo, e_hi


def _my_range(e_lo, e_hi, s):
  m_len = e_hi - e_lo
  my_lo = e_lo + (m_len * s) // 16
  my_hi = e_lo + (m_len * (s + 1)) // 16
  a_lo = (my_lo // 8) * 8
  nch = (my_hi - a_lo + (G - 1)) // G
  return my_lo, my_hi, a_lo, nch


def _prop_sc(u, src_s, dst_s, bounds):
  @functools.partial(
      pl.kernel,
      out_type=jax.ShapeDtypeStruct((NPAD, D), jnp.float32),
      mesh=_mesh,
      compiler_params=_sc_params,
      scratch_types=[
          pltpu.VMEM_SHARED((BLK + 8, D), jnp.float32),
          pltpu.VMEM((TPB // 5, D), jnp.float32),
          pltpu.VMEM((2, G), jnp.int32),
          pltpu.VMEM((2, G), jnp.int32),
          pltpu.VMEM((2, G), jnp.int32),
          pltpu.VMEM((2, G), jnp.int32),
          pltpu.VMEM((2, G, D), jnp.float32),
          pltpu.VMEM((16,), jnp.int32),
          pltpu.SemaphoreType.DMA,
          pltpu.SemaphoreType.DMA,
      ],
  )
  def k(u_hbm, src_hbm, dst_hbm, bnd_hbm, agg_hbm,
        acc, zbuf, srcv, dstv, csrc, cdst, rows, bvec, sem0, sem1):
    sems = (sem0, sem1)
    c = lax.axis_index("c")
    s = lax.axis_index("s")
    lane = lax.broadcasted_iota(jnp.int32, (16,), 0)
    zrows = TPB // 5

    @pl.loop(0, zrows)
    def _(r):
      for j in range(D // 16):
        zbuf[r, pl.ds(j * 16, 16)] = jnp.zeros((16,), jnp.float32)

    pltpu.sync_copy(bnd_hbm, bvec)

    for slot in range(SLOTS):
      blk_id = c * SLOTS + slot
      lo_row = blk_id * BLK
      e_lo, e_hi = _block_bounds(bvec, lane, blk_id)
      my_lo, my_hi, a_lo, nch = _my_range(e_lo, e_hi, s)

      for z in range(TPB // zrows):
        pltpu.sync_copy(zbuf, acc.at[pl.ds(s * TPB + z * zrows, zrows)])
      plsc.subcore_barrier()

      def pair(p, carry):
        for b in range(2):
          i = p * 2 + b

          @pl.when(i < nch)
          def _():
            @pl.when(i >= 2)
            def _():
              pltpu.make_async_copy(rows.at[b], acc.at[cdst.at[b]],
                                    sems[b]).wait()
            base = a_lo + i * G
            pltpu.sync_copy(src_hbm.at[pl.ds(base, G)], srcv.at[b])
            pltpu.sync_copy(dst_hbm.at[pl.ds(base, G)], dstv.at[b])
            _clean_indices(srcv, dstv, csrc, cdst, lane, base, my_lo, my_hi,
                           lo_row, b=b)
            pltpu.sync_copy(u_hbm.at[csrc.at[b]], rows.at[b])
            pltpu.async_copy(rows.at[b], acc.at[cdst.at[b]], sems[b],
                             add=True)
        return carry

      lax.fori_loop(0, (nch + 1) // 2, pair, 0)
      for b in range(2):
        @pl.when(nch > b)
        def _():
          pltpu.make_async_copy(rows.at[b], acc.at[cdst.at[b]],
                                sems[b]).wait()
      plsc.subcore_barrier()
      pltpu.sync_copy(acc.at[pl.ds(s * TPB, TPB)],
                      agg_hbm.at[pl.ds(lo_row + s * TPB, TPB)])

  return k(u, src_s, dst_s, bounds)


def _deg_sc(dst_s, bounds):
  @functools.partial(
      pl.kernel,
      out_type=jax.ShapeDtypeStruct((NPAD, 16), jnp.float32),
      mesh=_mesh,
      compiler_params=_sc_params,
      scratch_types=[
          pltpu.VMEM_SHARED((BLK + 8, 16), jnp.float32),
          pltpu.VMEM((G, 16), jnp.float32),
          pltpu.VMEM((TPB // 5, 16), jnp.float32),
          pltpu.VMEM((G,), jnp.int32),
          pltpu.VMEM((G,), jnp.int32),
          pltpu.VMEM((G,), jnp.int32),
          pltpu.VMEM((16,), jnp.int32),
      ],
  )
  def k(dst_hbm, bnd_hbm, deg_hbm, acc, ones, zbuf, srcv, dstv, cdst, bvec):
    c = lax.axis_index("c")
    s = lax.axis_index("s")
    lane = lax.broadcasted_iota(jnp.int32, (16,), 0)
    zrows = TPB // 5

    @pl.loop(0, G)
    def _(r):
      ones[r, pl.ds(0, 16)] = jnp.ones((16,), jnp.float32)

    @pl.loop(0, zrows)
    def _(r):
      zbuf[r, pl.ds(0, 16)] = jnp.zeros((16,), jnp.float32)

    pltpu.sync_copy(bnd_hbm, bvec)

    for slot in range(SLOTS):
      blk_id = c * SLOTS + slot
      lo_row = blk_id * BLK
      e_lo, e_hi = _block_bounds(bvec, lane, blk_id)
      my_lo, my_hi, a_lo, nch = _my_range(e_lo, e_hi, s)

      for z in range(TPB // zrows):
        pltpu.sync_copy(zbuf, acc.at[pl.ds(s * TPB + z * zrows, zrows)])
      plsc.subcore_barrier()

      def chunk(i, carry):
        base = a_lo + i * G
        pltpu.sync_copy(dst_hbm.at[pl.ds(base, G)], dstv)
        _clean_indices(dstv, dstv, srcv, cdst, lane, base, my_lo, my_hi,
                       lo_row)
        pltpu.sync_copy(ones, acc.at[cdst], add=True)
        return carry

      lax.fori_loop(0, nch, chunk, 0)
      plsc.subcore_barrier()
      pltpu.sync_copy(acc.at[pl.ds(s * TPB, TPB)],
                      deg_hbm.at[pl.ds(lo_row + s * TPB, TPB)])

  return k(dst_s, bounds)


def _mlp_tc(x, w1t, b1, w2t, b2):
  def body(x_ref, w1_ref, b1_ref, w2_ref, b2_ref, o_ref):
    h1 = jnp.dot(x_ref[...], w1_ref[...],
                 preferred_element_type=jnp.float32) + b1_ref[0:1, :]
    h1 = jnp.maximum(h1, 0.0)
    o_ref[...] = jnp.dot(h1, w2_ref[...],
                         preferred_element_type=jnp.float32) + b2_ref[0:1, :]

  return pl.pallas_call(
      body,
      grid=(NROW_BLOCKS,),
      in_specs=[
          pl.BlockSpec((RB, 128), lambda i: (i, 0)),
          pl.BlockSpec((128, 128), lambda i: (0, 0)),
          pl.BlockSpec((8, 128), lambda i: (0, 0)),
          pl.BlockSpec((128, D), lambda i: (0, 0)),
          pl.BlockSpec((8, D), lambda i: (0, 0)),
      ],
      out_specs=pl.BlockSpec((RB, D), lambda i: (i, 0)),
      out_shape=jax.ShapeDtypeStruct((N, D), jnp.float32),
  )(x, w1t, b1, w2t, b2)


def _prep_tc(h, deg16):
  def body(h_ref, d_ref, u_ref, c_ref, g_ref, q_ref):
    deg = d_ref[:, 0:1] + 1.0
    dinv = lax.rsqrt(deg)
    u0 = h_ref[...] * dinv
    u_ref[...] = u0
    c_ref[...] = jnp.broadcast_to((1.0 - ALPHA_C) * dinv * dinv,
                                  (RB, D))
    g_ref[...] = ALPHA_C * u0
    q_ref[...] = jnp.broadcast_to(deg * dinv, (RB, D))

  return pl.pallas_call(
      body,
      grid=(NROW_BLOCKS,),
      in_specs=[
          pl.BlockSpec((RB, D), lambda i: (i, 0)),
          pl.BlockSpec((RB, 16), lambda i: (i, 0)),
      ],
      out_specs=[
          pl.BlockSpec((RB, D), lambda i: (i, 0)),
          pl.BlockSpec((RB, D), lambda i: (i, 0)),
          pl.BlockSpec((RB, D), lambda i: (i, 0)),
          pl.BlockSpec((RB, D), lambda i: (i, 0)),
      ],
      out_shape=[jax.ShapeDtypeStruct((N, D), jnp.float32)] * 4,
  )(h, deg16)


def _update_tc(agg, u, c, g):
  def body(a_ref, u_ref, c_ref, g_ref, o_ref):
    o_ref[...] = c_ref[...] * (a_ref[...] + u_ref[...]) + g_ref[...]

  return pl.pallas_call(
      body,
      grid=(NROW_BLOCKS,),
      in_specs=[pl.BlockSpec((RB, D), lambda i: (i, 0))] * 4,
      out_specs=pl.BlockSpec((RB, D), lambda i: (i, 0)),
      out_shape=jax.ShapeDtypeStruct((N, D), jnp.float32),
  )(agg, u, c, g)


def _final_tc(u, dsq):
  def body(u_ref, q_ref, o_ref):
    z = u_ref[...] * q_ref[...]
    m = jnp.max(z, axis=1, keepdims=True)
    e = jnp.exp(z - m)
    ssum = jnp.sum(e, axis=1, keepdims=True)
    o_ref[...] = z - m - jnp.log(ssum)

  return pl.pallas_call(
      body,
      grid=(NROW_BLOCKS,),
      in_specs=[pl.BlockSpec((RB, D), lambda i: (i, 0))] * 2,
      out_specs=pl.BlockSpec((RB, D), lambda i: (i, 0)),
      out_shape=jax.ShapeDtypeStruct((N, D), jnp.float32),
  )(u, dsq)


def kernel(x, edge_index, W1, b1, W2, b2):
  src = edge_index[0]
  dst = edge_index[1]
  order = jnp.argsort(dst)
  src_s = jnp.concatenate([src[order], jnp.zeros((G,), jnp.int32)])
  dst_s = jnp.concatenate([dst[order], jnp.zeros((G,), jnp.int32)])
  edges = jnp.arange(16, dtype=jnp.int32) * BLK
  edges = jnp.minimum(edges, N)
  bounds = jnp.searchsorted(dst_s[:E], edges).astype(jnp.int32)

  h = _mlp_tc(x, W1.T, jnp.tile(b1[None, :], (8, 1)),
              W2.T, jnp.tile(b2[None, :], (8, 1)))
  deg16 = _deg_sc(dst_s, bounds)
  u, c, g, dsq = _prep_tc(h, deg16[:N])

  def step(_, u):
    agg = _prop_sc(u, src_s, dst_s, bounds)
    return _update_tc(agg[:N], u, c, g)

  u = lax.fori_loop(0, K_ITERS, step, u)
  return _final_tc(u, dsq)
